# baseline (speedup 1.0000x reference)
import jax
import jax.numpy as jnp
from jax import lax
from jax.experimental import pallas as pl
from jax.experimental.pallas import tpu as pltpu
from jax.experimental.pallas import tpu_sc as plsc

B = 4096
N_FIELDS = 26
VOCAB = 100000
D_EMB = 64

NUM_CORES = 2
NUM_SUBCORES = 16
NUM_WORKERS = NUM_CORES * NUM_SUBCORES
B_PER_W = B // NUM_WORKERS

ROWS_PER_TILE = 8
N_TILES = N_FIELDS * VOCAB // ROWS_PER_TILE
B_PER_CHUNK = 8
PAIRS_PER_CHUNK = B_PER_CHUNK * N_FIELDS
N_CHUNKS = B_PER_W // B_PER_CHUNK
PAIRS_PER_W = B_PER_W * N_FIELDS
LANES = 16
CGROUPS = D_EMB // LANES


def _sc_gather_sum(tile_ids, row_ids, tab4):
    mesh = plsc.VectorSubcoreMesh(
        core_axis_name="c", subcore_axis_name="s",
        num_cores=NUM_CORES, num_subcores=NUM_SUBCORES,
    )

    def body(tid_hbm, rid_hbm, tab_hbm, s_hbm, tid_v, rid_v, out_v, rows_v,
             gat_sem):
        cid = lax.axis_index("c")
        sid = lax.axis_index("s")
        wid = sid * NUM_CORES + cid
        pbase = wid * PAIRS_PER_W
        pltpu.sync_copy(tid_hbm.at[pl.ds(pbase, PAIRS_PER_W)],
                        tid_v.at[pl.ds(0, PAIRS_PER_W)])
        pltpu.sync_copy(rid_hbm.at[pl.ds(pbase, PAIRS_PER_W)],
                        rid_v.at[pl.ds(0, PAIRS_PER_W)])
        n_vec = (PAIRS_PER_CHUNK + LANES - 1) // LANES

        def chunk_body(c, carry):
            base = c * PAIRS_PER_CHUNK
            tvs = [tid_v[pl.ds(base + k * LANES, LANES)] for k in range(n_vec)]
            rvs = [rid_v[pl.ds(base + k * LANES, LANES)] for k in range(n_vec)]
            cps = []
            for i in range(PAIRS_PER_CHUNK):
                t = tvs[i // LANES][i % LANES]
                r = rvs[i // LANES][i % LANES]
                cps.append(pltpu.async_copy(
                    tab_hbm.at[t, pl.ds(r, 1)],
                    rows_v.at[pl.ds(i, 1)], gat_sem))
            for cp in cps:
                cp.wait()
            for bl in range(B_PER_CHUNK):
                accs = [jnp.zeros((LANES,), jnp.float32)
                        for _ in range(CGROUPS)]
                for f in range(N_FIELDS):
                    i = bl * N_FIELDS + f
                    for g in range(CGROUPS):
                        accs[g] = accs[g] + rows_v[i,
                                                   pl.ds(g * LANES, LANES)]
                ob = (c * B_PER_CHUNK + bl) * D_EMB
                for g in range(CGROUPS):
                    out_v[pl.ds(ob + g * LANES, LANES)] = accs[g]
            return carry

        lax.fori_loop(0, N_CHUNKS, chunk_body, 0)
        pltpu.sync_copy(out_v, s_hbm.at[pl.ds(wid * B_PER_W * D_EMB,
                                              B_PER_W * D_EMB)])

    call = pl.kernel(
        body,
        out_type=jax.ShapeDtypeStruct((B * D_EMB,), jnp.float32),
        mesh=mesh,
        name="sc_gather_sum",
        scratch_types=[
            pltpu.VMEM((PAIRS_PER_W + LANES,), jnp.int32),
            pltpu.VMEM((PAIRS_PER_W + LANES,), jnp.int32),
            pltpu.VMEM((B_PER_W * D_EMB,), jnp.float32),
            pltpu.VMEM((PAIRS_PER_CHUNK, D_EMB), jnp.float32),
            pltpu.SemaphoreType.DMA,
        ],
        compiler_params=pltpu.CompilerParams(use_tc_tiling_on_sc=True),
    )
    return call(tile_ids, row_ids, tab4)


def _tc_mlp(s, W1, b1, W2, b2):
    def body(s_ref, w1_ref, b1_ref, w2_ref, b2_ref, o_ref):
        h = jnp.dot(s_ref[...], w1_ref[...].T,
                    preferred_element_type=jnp.float32)
        h = h + jnp.float32(N_FIELDS) * b1_ref[...]
        o = jnp.dot(h, w2_ref[...], preferred_element_type=jnp.float32)
        o_ref[...] = o + jnp.float32(N_FIELDS) * b2_ref[0]

    w2p = jnp.zeros((32, 128), jnp.float32).at[:, 0].set(W2[0])
    out = pl.pallas_call(
        body,
        in_specs=[
            pl.BlockSpec(memory_space=pltpu.VMEM),
            pl.BlockSpec(memory_space=pltpu.VMEM),
            pl.BlockSpec(memory_space=pltpu.VMEM),
            pl.BlockSpec(memory_space=pltpu.VMEM),
            pl.BlockSpec(memory_space=pltpu.SMEM),
        ],
        out_shape=jax.ShapeDtypeStruct((B, 128), jnp.float32),
    )(s, W1, b1.reshape(1, 32), w2p, b2.reshape(1,))
    return out[:, :1]


def kernel(src, tables, W1, b1, W2, b2):
    src = src.astype(jnp.int32)
    offs = (jnp.arange(N_FIELDS, dtype=jnp.int32) * VOCAB)[None, :]
    flat = (src + offs).reshape(-1)
    tile_ids = flat >> 3
    row_ids = flat & 7
    tab4 = tables.reshape(N_TILES, ROWS_PER_TILE, D_EMB)
    s = _sc_gather_sum(tile_ids, row_ids, tab4)
    return _tc_mlp(s.reshape(B, D_EMB), W1, b1, W2, b2)


# speedup vs baseline: 1.4814x; 1.4814x over previous
import jax
import jax.numpy as jnp
from jax import lax
from jax.experimental import pallas as pl
from jax.experimental.pallas import tpu as pltpu
from jax.experimental.pallas import tpu_sc as plsc

B = 4096
N_FIELDS = 26
VOCAB = 100000
D_EMB = 64

NUM_CORES = 2
NUM_SUBCORES = 16
NUM_WORKERS = NUM_CORES * NUM_SUBCORES
D_PER_W = D_EMB // NUM_WORKERS
LANES = 16
B_GROUPS = B // LANES


def _sc_gather_sum_t(src_t, tab_t):
    mesh = plsc.VectorSubcoreMesh(
        core_axis_name="c", subcore_axis_name="s",
        num_cores=NUM_CORES, num_subcores=NUM_SUBCORES,
    )

    def body(src_hbm, tab_hbm, st_hbm, idx_v, row_v, acc_v):
        cid = lax.axis_index("c")
        sid = lax.axis_index("s")
        wid = sid * NUM_CORES + cid
        zero = jnp.zeros((LANES,), jnp.float32)

        def zbody(g, carry):
            acc_v[pl.ds(pl.multiple_of(g * LANES, LANES), LANES)] = zero
            return carry

        lax.fori_loop(0, D_PER_W * B_GROUPS, zbody, 0)

        def fbody(f, carry):
            pltpu.sync_copy(src_hbm.at[f], idx_v)
            for dd in range(D_PER_W):
                d = wid * D_PER_W + dd
                pltpu.sync_copy(tab_hbm.at[f, d], row_v)
                ab = dd * B

                def gbody(g, c2):
                    o = pl.multiple_of(g * LANES, LANES)
                    idx = idx_v[pl.ds(o, LANES)]
                    vals = plsc.load_gather(row_v, [idx])
                    a = pl.ds(ab + o, LANES)
                    acc_v[a] = acc_v[a] + vals
                    return c2

                lax.fori_loop(0, B_GROUPS, gbody, 0)
            return carry

        lax.fori_loop(0, N_FIELDS, fbody, 0)
        pltpu.sync_copy(acc_v,
                        st_hbm.at[pl.ds(wid * D_PER_W * B, D_PER_W * B)])

    call = pl.kernel(
        body,
        out_type=jax.ShapeDtypeStruct((D_EMB * B,), jnp.float32),
        mesh=mesh,
        name="sc_gather_sum_t",
        scratch_types=[
            pltpu.VMEM((B,), jnp.int32),
            pltpu.VMEM((VOCAB,), jnp.float32),
            pltpu.VMEM((D_PER_W * B,), jnp.float32),
        ],
        compiler_params=pltpu.CompilerParams(use_tc_tiling_on_sc=True,
                                             needs_layout_passes=False),
    )
    return call(src_t, tab_t)


def _tc_mlp_t(st, W1, b1, W2, b2):
    def body(st_ref, w1_ref, b1_ref, w2_ref, b2_ref, o_ref):
        g = jnp.dot(w2_ref[...], w1_ref[...],
                    preferred_element_type=jnp.float32)
        o = jnp.dot(g, st_ref[...], preferred_element_type=jnp.float32)
        c = jnp.sum(w2_ref[...][:1, :] * b1_ref[...]) + b2_ref[0]
        o_ref[...] = o + jnp.float32(N_FIELDS) * c

    w2p = jnp.zeros((128, 32), jnp.float32).at[0, :].set(W2[0])
    return pl.pallas_call(
        body,
        in_specs=[
            pl.BlockSpec(memory_space=pltpu.VMEM),
            pl.BlockSpec(memory_space=pltpu.VMEM),
            pl.BlockSpec(memory_space=pltpu.VMEM),
            pl.BlockSpec(memory_space=pltpu.VMEM),
            pl.BlockSpec(memory_space=pltpu.SMEM),
        ],
        out_shape=jax.ShapeDtypeStruct((128, B), jnp.float32),
    )(st, W1, b1.reshape(1, 32), w2p, b2.reshape(1,))


def kernel(src, tables, W1, b1, W2, b2):
    src_t = src.astype(jnp.int32).T
    tab_t = tables.transpose(0, 2, 1)
    st = _sc_gather_sum_t(src_t, tab_t)
    out_t = _tc_mlp_t(st.reshape(D_EMB, B), W1, b1, W2, b2)
    return out_t[0].reshape(B, 1)


# speedup vs baseline: 1.6004x; 1.0803x over previous
import jax
import jax.numpy as jnp
from jax import lax
from jax.experimental import pallas as pl
from jax.experimental.pallas import tpu as pltpu
from jax.experimental.pallas import tpu_sc as plsc

B = 4096
N_FIELDS = 26
VOCAB = 100000
D_EMB = 64

NUM_CORES = 2
NUM_SUBCORES = 16
NUM_WORKERS = NUM_CORES * NUM_SUBCORES
D_PER_W = D_EMB // NUM_WORKERS
LANES = 16
B_GROUPS = B // LANES


def _sc_gather_sum_t(src_t, tab_t):
    mesh = plsc.VectorSubcoreMesh(
        core_axis_name="c", subcore_axis_name="s",
        num_cores=NUM_CORES, num_subcores=NUM_SUBCORES,
    )

    def body(src_hbm, tab_hbm, st_hbm, idx_v, row_v, acc_v):
        cid = lax.axis_index("c")
        sid = lax.axis_index("s")
        wid = sid * NUM_CORES + cid
        zero = jnp.zeros((LANES,), jnp.float32)

        def zbody(g, carry):
            acc_v[pl.ds(pl.multiple_of(g * LANES, LANES), LANES)] = zero
            return carry

        lax.fori_loop(0, D_PER_W * B_GROUPS, zbody, 0)

        def fbody(f, carry):
            pltpu.sync_copy(src_hbm.at[f], idx_v)
            for dd in range(D_PER_W):
                d = wid * D_PER_W + dd
                pltpu.sync_copy(tab_hbm.at[f, d], row_v)
                ab = dd * B

                def gbody(g8, c2):
                    ob = pl.multiple_of(g8 * (8 * LANES), 8 * LANES)
                    for u in range(8):
                        o = ob + u * LANES
                        idx = idx_v[pl.ds(o, LANES)]
                        vals = plsc.load_gather(row_v, [idx])
                        a = pl.ds(ab + o, LANES)
                        acc_v[a] = acc_v[a] + vals
                    return c2

                lax.fori_loop(0, B_GROUPS // 8, gbody, 0)
            return carry

        lax.fori_loop(0, N_FIELDS, fbody, 0)
        pltpu.sync_copy(acc_v,
                        st_hbm.at[pl.ds(wid * D_PER_W * B, D_PER_W * B)])

    call = pl.kernel(
        body,
        out_type=jax.ShapeDtypeStruct((D_EMB * B,), jnp.float32),
        mesh=mesh,
        name="sc_gather_sum_t",
        scratch_types=[
            pltpu.VMEM((B,), jnp.int32),
            pltpu.VMEM((VOCAB,), jnp.float32),
            pltpu.VMEM((D_PER_W * B,), jnp.float32),
        ],
        compiler_params=pltpu.CompilerParams(use_tc_tiling_on_sc=True,
                                             needs_layout_passes=False),
    )
    return call(src_t, tab_t)


def _tc_mlp_t(st, W1, b1, W2, b2):
    def body(st_ref, w1_ref, b1_ref, w2_ref, b2_ref, o_ref):
        g = jnp.dot(w2_ref[...], w1_ref[...],
                    preferred_element_type=jnp.float32)
        o = jnp.dot(g, st_ref[...], preferred_element_type=jnp.float32)
        c = jnp.sum(w2_ref[...][:1, :] * b1_ref[...]) + b2_ref[0]
        o_ref[...] = o + jnp.float32(N_FIELDS) * c

    w2p = jnp.zeros((128, 32), jnp.float32).at[0, :].set(W2[0])
    return pl.pallas_call(
        body,
        in_specs=[
            pl.BlockSpec(memory_space=pltpu.VMEM),
            pl.BlockSpec(memory_space=pltpu.VMEM),
            pl.BlockSpec(memory_space=pltpu.VMEM),
            pl.BlockSpec(memory_space=pltpu.VMEM),
            pl.BlockSpec(memory_space=pltpu.SMEM),
        ],
        out_shape=jax.ShapeDtypeStruct((128, B), jnp.float32),
    )(st, W1, b1.reshape(1, 32), w2p, b2.reshape(1,))


def kernel(src, tables, W1, b1, W2, b2):
    src_t = src.astype(jnp.int32).T
    tab_t = tables.transpose(0, 2, 1)
    st = _sc_gather_sum_t(src_t, tab_t)
    out_t = _tc_mlp_t(st.reshape(D_EMB, B), W1, b1, W2, b2)
    return out_t[0].reshape(B, 1)


# speedup vs baseline: 1.6889x; 1.0553x over previous
import jax
import jax.numpy as jnp
from jax import lax
from jax.experimental import pallas as pl
from jax.experimental.pallas import tpu as pltpu
from jax.experimental.pallas import tpu_sc as plsc

B = 4096
N_FIELDS = 26
VOCAB = 100000
D_EMB = 64

NUM_CORES = 2
NUM_SUBCORES = 16
NUM_WORKERS = NUM_CORES * NUM_SUBCORES
D_PER_W = D_EMB // NUM_WORKERS
LANES = 16
B_GROUPS = B // LANES


def _sc_gather_sum_t(src_t, tab_t):
    mesh = plsc.VectorSubcoreMesh(
        core_axis_name="c", subcore_axis_name="s",
        num_cores=NUM_CORES, num_subcores=NUM_SUBCORES,
    )

    def body(src_hbm, tab_hbm, st_hbm, idx_v, row_v, acc_v):
        cid = lax.axis_index("c")
        sid = lax.axis_index("s")
        wid = sid * NUM_CORES + cid
        zero = jnp.zeros((LANES,), jnp.float32)

        def zbody(g, carry):
            acc_v[pl.ds(pl.multiple_of(g * LANES, LANES), LANES)] = zero
            return carry

        lax.fori_loop(0, D_PER_W * B_GROUPS, zbody, 0)

        def fbody(f, carry):
            pltpu.sync_copy(src_hbm.at[f], idx_v)
            for dd in range(D_PER_W):
                d = wid * D_PER_W + dd
                pltpu.sync_copy(tab_hbm.at[f, d], row_v)
                ab = dd * B

                def gbody(g8, c2):
                    ob = pl.multiple_of(g8 * (8 * LANES), 8 * LANES)
                    for u in range(8):
                        o = ob + u * LANES
                        idx = idx_v[pl.ds(o, LANES)]
                        vals = plsc.load_gather(row_v, [idx])
                        plsc.addupdate(acc_v.at[pl.ds(ab + o, LANES)], vals)
                    return c2

                lax.fori_loop(0, B_GROUPS // 8, gbody, 0)
            return carry

        lax.fori_loop(0, N_FIELDS, fbody, 0)
        pltpu.sync_copy(acc_v,
                        st_hbm.at[pl.ds(wid * D_PER_W * B, D_PER_W * B)])

    call = pl.kernel(
        body,
        out_type=jax.ShapeDtypeStruct((D_EMB * B,), jnp.float32),
        mesh=mesh,
        name="sc_gather_sum_t",
        scratch_types=[
            pltpu.VMEM((B,), jnp.int32),
            pltpu.VMEM((VOCAB,), jnp.float32),
            pltpu.VMEM((D_PER_W * B,), jnp.float32),
        ],
        compiler_params=pltpu.CompilerParams(use_tc_tiling_on_sc=True,
                                             needs_layout_passes=False),
    )
    return call(src_t, tab_t)


def _tc_mlp_t(st, W1, b1, W2, b2):
    def body(st_ref, w1_ref, b1_ref, w2_ref, b2_ref, o_ref):
        g = jnp.dot(w2_ref[...], w1_ref[...],
                    preferred_element_type=jnp.float32)
        o = jnp.dot(g, st_ref[...], preferred_element_type=jnp.float32)
        c = jnp.sum(w2_ref[...][:1, :] * b1_ref[...]) + b2_ref[0]
        o_ref[...] = o + jnp.float32(N_FIELDS) * c

    w2p = jnp.zeros((128, 32), jnp.float32).at[0, :].set(W2[0])
    return pl.pallas_call(
        body,
        in_specs=[
            pl.BlockSpec(memory_space=pltpu.VMEM),
            pl.BlockSpec(memory_space=pltpu.VMEM),
            pl.BlockSpec(memory_space=pltpu.VMEM),
            pl.BlockSpec(memory_space=pltpu.VMEM),
            pl.BlockSpec(memory_space=pltpu.SMEM),
        ],
        out_shape=jax.ShapeDtypeStruct((128, B), jnp.float32),
    )(st, W1, b1.reshape(1, 32), w2p, b2.reshape(1,))


def kernel(src, tables, W1, b1, W2, b2):
    src_t = src.astype(jnp.int32).T
    tab_t = tables.transpose(0, 2, 1)
    st = _sc_gather_sum_t(src_t, tab_t)
    out_t = _tc_mlp_t(st.reshape(D_EMB, B), W1, b1, W2, b2)
    return out_t[0].reshape(B, 1)
